# baseline (speedup 1.0000x reference)
import functools

import jax
import jax.numpy as jnp
from jax import lax
from jax.experimental import pallas as pl
from jax.experimental.pallas import tpu as pltpu
from jax.experimental.pallas import tpu_sc as plsc

NUM_CORES = 2
NUM_SUBCORES = 16
NW = NUM_CORES * NUM_SUBCORES

CHUNK = 512


def _emb_kernel(n_chunks, b_per_w, idx_hbm, table_hbm, out_hbm,
                idx_v, rows_v, sem_g):
    wid = lax.axis_index("s") * NUM_CORES + lax.axis_index("c")
    base = wid * b_per_w

    def body(i, _):
        off = pl.multiple_of(base + i * CHUNK, CHUNK)
        pltpu.sync_copy(idx_hbm.at[pl.ds(off, CHUNK)], idx_v)
        pltpu.async_copy(table_hbm.at[idx_v], rows_v, sem_g).wait()
        pltpu.sync_copy(rows_v, out_hbm.at[pl.ds(off, CHUNK)])
        return _

    lax.fori_loop(0, n_chunks, body, None)


def kernel(token_ids, weight):
    batch, seq = token_ids.shape
    n, d = weight.shape
    b = batch * seq
    assert b % (NW * CHUNK) == 0
    b_per_w = b // NW
    n_chunks = b_per_w // CHUNK

    flat_ids = token_ids.reshape(b).astype(jnp.int32)

    mesh = plsc.VectorSubcoreMesh(
        core_axis_name="c", subcore_axis_name="s",
        num_cores=NUM_CORES, num_subcores=NUM_SUBCORES)

    run = pl.kernel(
        functools.partial(_emb_kernel, n_chunks, b_per_w),
        out_type=jax.ShapeDtypeStruct((b, d), jnp.float32),
        mesh=mesh,
        scratch_types=[
            pltpu.VMEM((CHUNK,), jnp.int32),
            pltpu.VMEM((CHUNK, d), jnp.float32),
            pltpu.SemaphoreType.DMA,
        ],
        compiler_params=pltpu.CompilerParams(use_tc_tiling_on_sc=False),
    )
    out = run(flat_ids, weight)
    return out.reshape(batch, seq, d)


# speedup vs baseline: 1.0475x; 1.0475x over previous
import functools

import jax
import jax.numpy as jnp
from jax import lax
from jax.experimental import pallas as pl
from jax.experimental.pallas import tpu as pltpu
from jax.experimental.pallas import tpu_sc as plsc

NUM_CORES = 2
NUM_SUBCORES = 16
NW = NUM_CORES * NUM_SUBCORES

CHUNK = 512
NBUF = 2


def _emb_kernel(n_chunks, b_per_w, d, idx_hbm, table_hbm, out_hbm,
                idx_v, rows_v, sems_g, sems_w):
    wid = lax.axis_index("s") * NUM_CORES + lax.axis_index("c")
    base = wid * b_per_w
    n_groups = n_chunks // NBUF

    pltpu.sync_copy(idx_hbm.at[wid], idx_v)

    def gather_desc(i, s):
        return pltpu.make_async_copy(table_hbm.at[idx_v.at[i]], rows_v[s],
                                     sems_g[s])

    def write_desc(i, s):
        off = pl.multiple_of(base + i * CHUNK, CHUNK)
        return pltpu.make_async_copy(rows_v[s], out_hbm.at[pl.ds(off, CHUNK)],
                                     sems_w[s])

    for s in range(NBUF):
        gather_desc(s, s).start()

    def body(j, _):
        for s in range(NBUF):
            i = j * NBUF + s
            gather_desc(i - NBUF, s).wait()
            write_desc(i - NBUF, s).start()
            write_desc(i - NBUF, s).wait()
            gather_desc(i, s).start()
        return _

    lax.fori_loop(1, n_groups, body, None)

    for s in range(NBUF):
        i = (n_groups - 1) * NBUF + s
        gather_desc(i, s).wait()
        write_desc(i, s).start()
    for s in range(NBUF):
        i = (n_groups - 1) * NBUF + s
        write_desc(i, s).wait()


def kernel(token_ids, weight):
    batch, seq = token_ids.shape
    n, d = weight.shape
    b = batch * seq
    assert b % (NW * CHUNK * NBUF) == 0
    b_per_w = b // NW
    n_chunks = b_per_w // CHUNK

    flat_ids = token_ids.reshape(NW, n_chunks, CHUNK).astype(jnp.int32)

    mesh = plsc.VectorSubcoreMesh(
        core_axis_name="c", subcore_axis_name="s",
        num_cores=NUM_CORES, num_subcores=NUM_SUBCORES)

    run = pl.kernel(
        functools.partial(_emb_kernel, n_chunks, b_per_w, d),
        out_type=jax.ShapeDtypeStruct((b, d), jnp.float32),
        mesh=mesh,
        scratch_types=[
            pltpu.VMEM((n_chunks, CHUNK), jnp.int32),
            [pltpu.VMEM((CHUNK, d), jnp.float32) for _ in range(NBUF)],
            [pltpu.SemaphoreType.DMA for _ in range(NBUF)],
            [pltpu.SemaphoreType.DMA for _ in range(NBUF)],
        ],
        compiler_params=pltpu.CompilerParams(use_tc_tiling_on_sc=False),
    )
    out = run(flat_ids, weight)
    return out.reshape(batch, seq, d)
